# baseline (speedup 1.0000x reference)
import functools

import jax
import jax.numpy as jnp
from jax import lax
from jax.experimental import pallas as pl
from jax.experimental.pallas import tpu as pltpu
from jax.experimental.pallas import tpu_sc as plsc

N_NODES = 1000000
EMBED_DIM = 64
BATCH = 16384

_INFO = plsc.get_sparse_core_info()
_NC = _INFO.num_cores
_NS = _INFO.num_subcores
_NW = _NC * _NS
_ROWS_PER_W = BATCH // _NW
_LANES = 16
_DGROUPS = EMBED_DIM // _LANES
_NBLOCKS = _ROWS_PER_W // _LANES


def _sc_kernel(head_hbm, tail_hbm, table_hbm, rel_hbm, out_hbm,
               hidx_v, tidx_v, hrows_v, trows_v, rel_v, out_v, sem):
    wid = lax.axis_index("s") * _NC + lax.axis_index("c")
    base = wid * _ROWS_PER_W

    pltpu.sync_copy(rel_hbm, rel_v)
    pltpu.sync_copy(head_hbm.at[pl.ds(base, _ROWS_PER_W)], hidx_v)
    pltpu.sync_copy(tail_hbm.at[pl.ds(base, _ROWS_PER_W)], tidx_v)

    def fire_group(g, carry):
        row0 = g * _LANES
        pair0 = g * (_LANES // 2)
        hv = hidx_v[pl.ds(row0, _LANES)]
        tv = tidx_v[pl.ds(row0, _LANES)]
        for i in range(_LANES):
            dst_row = pair0 + i // 2
            dst_col = (i % 2) * EMBED_DIM
            pltpu.async_copy(
                table_hbm.at[hv[i]],
                hrows_v.at[dst_row, pl.ds(dst_col, EMBED_DIM)], sem)
            pltpu.async_copy(
                table_hbm.at[tv[i]],
                trows_v.at[dst_row, pl.ds(dst_col, EMBED_DIM)], sem)
        return carry

    lax.fori_loop(0, _NBLOCKS, fire_group, 0)

    pltpu.make_async_copy(table_hbm.at[pl.ds(0, _ROWS_PER_W)], hrows_v,
                          sem).wait()
    pltpu.make_async_copy(table_hbm.at[pl.ds(0, _ROWS_PER_W)], trows_v,
                          sem).wait()

    rel_regs = [rel_v[pl.ds(j * _LANES, _LANES)] for j in range(_DGROUPS)]

    lane_iota = lax.iota(jnp.int32, _LANES)
    onehot = [(lane_iota == i).astype(jnp.float32) for i in range(_LANES)]

    def block_body(k, carry):
        pair0 = k * (_LANES // 2)
        scores = jnp.zeros((_LANES,), jnp.float32)
        for i in range(_LANES):
            brow = pair0 + i // 2
            bcol = (i % 2) * EMBED_DIM
            acc = (hrows_v[brow, pl.ds(bcol, _LANES)]
                   * trows_v[brow, pl.ds(bcol, _LANES)] * rel_regs[0])
            for j in range(1, _DGROUPS):
                acc = acc + (hrows_v[brow, pl.ds(bcol + j * _LANES, _LANES)]
                             * trows_v[brow, pl.ds(bcol + j * _LANES, _LANES)]
                             * rel_regs[j])
            scores = scores + jnp.sum(acc) * onehot[i]
        out_v[pl.ds(k * _LANES, _LANES)] = scores
        return carry

    lax.fori_loop(0, _NBLOCKS, block_body, 0)

    pltpu.sync_copy(out_v, out_hbm.at[pl.ds(base, _ROWS_PER_W)])


_TBLK = 2048


def _tc_transpose_body(x_ref, o_ref):
    o_ref[...] = x_ref[...].T


def _tc_transpose(table_t):
    grid = (N_NODES + _TBLK - 1) // _TBLK
    return pl.pallas_call(
        _tc_transpose_body,
        grid=(grid,),
        in_specs=[pl.BlockSpec((EMBED_DIM, _TBLK), lambda i: (0, i))],
        out_specs=pl.BlockSpec((_TBLK, EMBED_DIM), lambda i: (i, 0)),
        out_shape=jax.ShapeDtypeStruct((N_NODES, EMBED_DIM), jnp.float32),
    )(table_t)


@jax.jit
def _run(head_idx, tail_idx, table_t, rel):
    table = _tc_transpose(table_t)
    mesh = plsc.VectorSubcoreMesh(core_axis_name="c", subcore_axis_name="s")
    kern = functools.partial(
        pl.kernel,
        mesh=mesh,
        compiler_params=pltpu.CompilerParams(needs_layout_passes=False),
        out_type=jax.ShapeDtypeStruct((BATCH,), jnp.float32),
        scratch_types=[
            pltpu.VMEM((_ROWS_PER_W,), jnp.int32),
            pltpu.VMEM((_ROWS_PER_W,), jnp.int32),
            pltpu.VMEM((_ROWS_PER_W // 2, 2 * EMBED_DIM), jnp.float32),
            pltpu.VMEM((_ROWS_PER_W // 2, 2 * EMBED_DIM), jnp.float32),
            pltpu.VMEM((EMBED_DIM,), jnp.float32),
            pltpu.VMEM((_ROWS_PER_W,), jnp.float32),
            pltpu.SemaphoreType.DMA,
        ],
    )(_sc_kernel)
    return kern(head_idx, tail_idx, table, rel)


def kernel(head_indices, tail_indices, node_embedding, relation_vector):
    return _run(head_indices.astype(jnp.int32),
                tail_indices.astype(jnp.int32),
                node_embedding.T, relation_vector)


# speedup vs baseline: 1.7972x; 1.7972x over previous
import functools

import jax
import jax.numpy as jnp
from jax import lax
from jax.experimental import pallas as pl
from jax.experimental.pallas import tpu as pltpu
from jax.experimental.pallas import tpu_sc as plsc

N_NODES = 1000000
EMBED_DIM = 64
BATCH = 16384

_INFO = plsc.get_sparse_core_info()
_NC = _INFO.num_cores
_NS = _INFO.num_subcores
_NW = _NC * _NS
_ROWS_PER_W = BATCH // _NW
_LANES = 16
_DGROUPS = EMBED_DIM // _LANES
_NBLOCKS = _ROWS_PER_W // _LANES


def _sc_kernel(head_hbm, tail_hbm, table_hbm, rel_hbm, out_hbm,
               hidx_v, tidx_v, hrows_v, trows_v, rel_v, out_v, sem):
    wid = lax.axis_index("s") * _NC + lax.axis_index("c")
    base = wid * _ROWS_PER_W

    pltpu.sync_copy(rel_hbm, rel_v)
    pltpu.sync_copy(head_hbm.at[pl.ds(base, _ROWS_PER_W)], hidx_v)
    pltpu.sync_copy(tail_hbm.at[pl.ds(base, _ROWS_PER_W)], tidx_v)

    def fire_group(g, carry):
        row0 = g * _LANES
        pair0 = g * (_LANES // 2)
        hv = hidx_v[pl.ds(row0, _LANES)]
        tv = tidx_v[pl.ds(row0, _LANES)]
        for i in range(_LANES):
            dst_row = pair0 + i // 2
            dst_col = (i % 2) * EMBED_DIM
            pltpu.async_copy(
                table_hbm.at[hv[i]],
                hrows_v.at[dst_row, pl.ds(dst_col, EMBED_DIM)], sem)
            pltpu.async_copy(
                table_hbm.at[tv[i]],
                trows_v.at[dst_row, pl.ds(dst_col, EMBED_DIM)], sem)
        return carry

    lax.fori_loop(0, _NBLOCKS, fire_group, 0)

    pltpu.make_async_copy(table_hbm.at[pl.ds(0, _ROWS_PER_W)], hrows_v,
                          sem).wait()
    pltpu.make_async_copy(table_hbm.at[pl.ds(0, _ROWS_PER_W)], trows_v,
                          sem).wait()

    rel_regs = [rel_v[pl.ds(j * _LANES, _LANES)] for j in range(_DGROUPS)]

    lane_iota = lax.iota(jnp.int32, _LANES)
    onehot = [(lane_iota == i).astype(jnp.float32) for i in range(_LANES)]

    def block_body(k, carry):
        pair0 = k * (_LANES // 2)
        scores = jnp.zeros((_LANES,), jnp.float32)
        for i in range(_LANES):
            brow = pair0 + i // 2
            bcol = (i % 2) * EMBED_DIM
            acc = (hrows_v[brow, pl.ds(bcol, _LANES)]
                   * trows_v[brow, pl.ds(bcol, _LANES)] * rel_regs[0])
            for j in range(1, _DGROUPS):
                acc = acc + (hrows_v[brow, pl.ds(bcol + j * _LANES, _LANES)]
                             * trows_v[brow, pl.ds(bcol + j * _LANES, _LANES)]
                             * rel_regs[j])
            scores = scores + jnp.sum(acc) * onehot[i]
        out_v[pl.ds(k * _LANES, _LANES)] = scores
        return carry

    lax.fori_loop(0, _NBLOCKS, block_body, 0)

    pltpu.sync_copy(out_v, out_hbm.at[pl.ds(base, _ROWS_PER_W)])


_TBLK = 16384


def _tc_transpose_body(x_ref, o_ref):
    o_ref[...] = x_ref[...].T


def _tc_transpose(table_t):
    grid = (N_NODES + _TBLK - 1) // _TBLK
    return pl.pallas_call(
        _tc_transpose_body,
        grid=(grid,),
        in_specs=[pl.BlockSpec((EMBED_DIM, _TBLK), lambda i: (0, i))],
        out_specs=pl.BlockSpec((_TBLK, EMBED_DIM), lambda i: (i, 0)),
        out_shape=jax.ShapeDtypeStruct((N_NODES, EMBED_DIM), jnp.float32),
    )(table_t)


@jax.jit
def _run(head_idx, tail_idx, table_t, rel):
    table = _tc_transpose(table_t)
    mesh = plsc.VectorSubcoreMesh(core_axis_name="c", subcore_axis_name="s")
    kern = functools.partial(
        pl.kernel,
        mesh=mesh,
        compiler_params=pltpu.CompilerParams(needs_layout_passes=False),
        out_type=jax.ShapeDtypeStruct((BATCH,), jnp.float32),
        scratch_types=[
            pltpu.VMEM((_ROWS_PER_W,), jnp.int32),
            pltpu.VMEM((_ROWS_PER_W,), jnp.int32),
            pltpu.VMEM((_ROWS_PER_W // 2, 2 * EMBED_DIM), jnp.float32),
            pltpu.VMEM((_ROWS_PER_W // 2, 2 * EMBED_DIM), jnp.float32),
            pltpu.VMEM((EMBED_DIM,), jnp.float32),
            pltpu.VMEM((_ROWS_PER_W,), jnp.float32),
            pltpu.SemaphoreType.DMA,
        ],
    )(_sc_kernel)
    return kern(head_idx, tail_idx, table, rel)


def kernel(head_indices, tail_indices, node_embedding, relation_vector):
    return _run(head_indices.astype(jnp.int32),
                tail_indices.astype(jnp.int32),
                node_embedding.T, relation_vector)


# speedup vs baseline: 1.8365x; 1.0219x over previous
import functools

import jax
import jax.numpy as jnp
from jax import lax
from jax.experimental import pallas as pl
from jax.experimental.pallas import tpu as pltpu
from jax.experimental.pallas import tpu_sc as plsc

N_NODES = 1000000
EMBED_DIM = 64
BATCH = 16384

_INFO = plsc.get_sparse_core_info()
_NC = _INFO.num_cores
_NS = _INFO.num_subcores
_NW = _NC * _NS
_ROWS_PER_W = BATCH // _NW
_LANES = 16
_DGROUPS = EMBED_DIM // _LANES
_NBLOCKS = _ROWS_PER_W // _LANES


def _sc_kernel(head_hbm, tail_hbm, table_hbm, rel_hbm, out_hbm,
               hidx_v, tidx_v, hrows_v, trows_v, rel_v, out_v, sem):
    wid = lax.axis_index("s") * _NC + lax.axis_index("c")
    base = wid * _ROWS_PER_W

    pltpu.sync_copy(rel_hbm, rel_v)
    pltpu.sync_copy(head_hbm.at[pl.ds(base, _ROWS_PER_W)], hidx_v)
    pltpu.sync_copy(tail_hbm.at[pl.ds(base, _ROWS_PER_W)], tidx_v)

    def fire_group(g, carry):
        row0 = g * _LANES
        pair0 = g * (_LANES // 2)
        hv = hidx_v[pl.ds(row0, _LANES)]
        tv = tidx_v[pl.ds(row0, _LANES)]
        for i in range(_LANES):
            dst_row = pair0 + i // 2
            dst_col = (i % 2) * EMBED_DIM
            pltpu.async_copy(
                table_hbm.at[hv[i]],
                hrows_v.at[dst_row, pl.ds(dst_col, EMBED_DIM)], sem)
            pltpu.async_copy(
                table_hbm.at[tv[i]],
                trows_v.at[dst_row, pl.ds(dst_col, EMBED_DIM)], sem)
        return carry

    lax.fori_loop(0, _NBLOCKS, fire_group, 0)

    pltpu.make_async_copy(table_hbm.at[pl.ds(0, _ROWS_PER_W)], hrows_v,
                          sem).wait()
    pltpu.make_async_copy(table_hbm.at[pl.ds(0, _ROWS_PER_W)], trows_v,
                          sem).wait()

    rel_regs = [rel_v[pl.ds(j * _LANES, _LANES)] for j in range(_DGROUPS)]

    lane_iota = lax.iota(jnp.int32, _LANES)
    onehot = [(lane_iota == i).astype(jnp.float32) for i in range(_LANES)]

    def block_body(k, carry):
        pair0 = k * (_LANES // 2)
        scores = jnp.zeros((_LANES,), jnp.float32)
        for i in range(_LANES):
            brow = pair0 + i // 2
            bcol = (i % 2) * EMBED_DIM
            acc = (hrows_v[brow, pl.ds(bcol, _LANES)]
                   * trows_v[brow, pl.ds(bcol, _LANES)] * rel_regs[0])
            for j in range(1, _DGROUPS):
                acc = acc + (hrows_v[brow, pl.ds(bcol + j * _LANES, _LANES)]
                             * trows_v[brow, pl.ds(bcol + j * _LANES, _LANES)]
                             * rel_regs[j])
            scores = scores + jnp.sum(acc) * onehot[i]
        out_v[pl.ds(k * _LANES, _LANES)] = scores
        return carry

    lax.fori_loop(0, _NBLOCKS, block_body, 0)

    pltpu.sync_copy(out_v, out_hbm.at[pl.ds(base, _ROWS_PER_W)])


_TBLK = 32768


def _tc_transpose_body(x_ref, o_ref):
    o_ref[...] = x_ref[...].T


def _tc_transpose(table_t):
    grid = (N_NODES + _TBLK - 1) // _TBLK
    return pl.pallas_call(
        _tc_transpose_body,
        grid=(grid,),
        in_specs=[pl.BlockSpec((EMBED_DIM, _TBLK), lambda i: (0, i))],
        out_specs=pl.BlockSpec((_TBLK, EMBED_DIM), lambda i: (i, 0)),
        out_shape=jax.ShapeDtypeStruct((N_NODES, EMBED_DIM), jnp.float32),
    )(table_t)


@jax.jit
def _run(head_idx, tail_idx, table_t, rel):
    table = _tc_transpose(table_t)
    mesh = plsc.VectorSubcoreMesh(core_axis_name="c", subcore_axis_name="s")
    kern = functools.partial(
        pl.kernel,
        mesh=mesh,
        compiler_params=pltpu.CompilerParams(needs_layout_passes=False),
        out_type=jax.ShapeDtypeStruct((BATCH,), jnp.float32),
        scratch_types=[
            pltpu.VMEM((_ROWS_PER_W,), jnp.int32),
            pltpu.VMEM((_ROWS_PER_W,), jnp.int32),
            pltpu.VMEM((_ROWS_PER_W // 2, 2 * EMBED_DIM), jnp.float32),
            pltpu.VMEM((_ROWS_PER_W // 2, 2 * EMBED_DIM), jnp.float32),
            pltpu.VMEM((EMBED_DIM,), jnp.float32),
            pltpu.VMEM((_ROWS_PER_W,), jnp.float32),
            pltpu.SemaphoreType.DMA,
        ],
    )(_sc_kernel)
    return kern(head_idx, tail_idx, table, rel)


def kernel(head_indices, tail_indices, node_embedding, relation_vector):
    return _run(head_indices.astype(jnp.int32),
                tail_indices.astype(jnp.int32),
                node_embedding.T, relation_vector)
